# Initial kernel scaffold; baseline (speedup 1.0000x reference)
import jax
import jax.numpy as jnp
from jax.experimental import pallas as pl


def kernel(x, phys_pos, latent_tokens_pos, edge_index, W_lift, b_lift, W_k1, b_k1, W_out, b_out):
    raise NotImplementedError("write your pallas kernel here")


# speedup vs baseline: 1.5225x; 1.5225x over previous
import functools

import jax
import jax.numpy as jnp
from jax import lax
from jax.experimental import pallas as pl
from jax.experimental.pallas import tpu as pltpu
from jax.experimental.pallas import tpu_sc as plsc

N = 10000
E = 320000
M = 2048
D = 128
H = 128

NC = 2
NS = 16
NW = NC * NS
EPW = E // NW
K = 80
NCHUNK = EPW // K

_GK1 = 1.5957691216057308
_GK2 = _GK1 * 0.044715


def _src_table_body(x_ref, p8_ref, u8_ref, wl_ref, bl_ref, out_ref):
    out_ref[:, :H] = jnp.dot(p8_ref[...], u8_ref[...],
                             preferred_element_type=jnp.float32)
    out_ref[:, H:] = jnp.dot(x_ref[...], wl_ref[...],
                             preferred_element_type=jnp.float32) + bl_ref[...]


def _dst_table_body(l8_ref, v8_ref, bk_ref, out_ref):
    out_ref[...] = jnp.dot(l8_ref[...], v8_ref[...],
                           preferred_element_type=jnp.float32) + bk_ref[...]


def _edge_body(srct_hbm, dstt_hbm, src_hbm, dst_hbm, out_hbm, cnt_hbm,
               sidx_v, didx_v, gath_v, drow_v, msg_v, ones_v, zbuf_v,
               acc_sh, cnt_sh):
    c = lax.axis_index("c")
    s = lax.axis_index("s")
    wid = s * NC + c

    def zfill(i, carry):
        r = i // (H // 16)
        col = (i % (H // 16)) * 16
        zbuf_v[r, pl.ds(col, 16)] = jnp.zeros((16,), jnp.float32)
        return carry
    lax.fori_loop(0, 16 * (H // 16), zfill, 0)

    def zcopy(r, carry):
        pltpu.sync_copy(zbuf_v, acc_sh.at[pl.ds(s * (M // NS) + r * 16, 16)])
        pltpu.sync_copy(zbuf_v, cnt_sh.at[pl.ds(s * (M // NS) + r * 16, 16)])
        return carry
    lax.fori_loop(0, (M // NS) // 16, zcopy, 0)

    def ofill(i, carry):
        r = i // (H // 16)
        col = (i % (H // 16)) * 16
        ones_v[r, pl.ds(col, 16)] = jnp.ones((16,), jnp.float32)
        return carry
    lax.fori_loop(0, K * (H // 16), ofill, 0)

    plsc.subcore_barrier()

    base_e = wid * EPW

    def chunk(i, carry):
        base = base_e + i * K
        pltpu.sync_copy(src_hbm.at[pl.ds(base, K)], sidx_v)
        pltpu.sync_copy(dst_hbm.at[pl.ds(base, K)], didx_v)
        pltpu.sync_copy(srct_hbm.at[sidx_v], gath_v)
        pltpu.sync_copy(dstt_hbm.at[didx_v], drow_v)

        def edge(e, carry2):
            for j in range(H // 16):
                a = gath_v[e, pl.ds(j * 16, 16)]
                hh = gath_v[e, pl.ds(H + j * 16, 16)]
                b = drow_v[e, pl.ds(j * 16, 16)]
                x0 = a + b
                sq = x0 * x0
                zneg = x0 * (sq * (-_GK2) + (-_GK1))
                den = jnp.exp(zneg) + 1.0
                msg_v[e, pl.ds(j * 16, 16)] = (x0 * hh) / den
            return carry2
        lax.fori_loop(0, K, edge, 0)

        pltpu.sync_copy(msg_v, acc_sh.at[didx_v], add=True)
        pltpu.sync_copy(ones_v, cnt_sh.at[didx_v], add=True)
        return carry
    lax.fori_loop(0, NCHUNK, chunk, 0)

    plsc.subcore_barrier()

    rows = M // NS
    pltpu.sync_copy(acc_sh.at[pl.ds(s * rows, rows)],
                    out_hbm.at[c, pl.ds(s * rows, rows)])
    pltpu.sync_copy(cnt_sh.at[pl.ds(s * rows, rows)],
                    cnt_hbm.at[c, pl.ds(s * rows, rows)])


_edge_kernel = functools.partial(
    pl.kernel,
    out_type=[jax.ShapeDtypeStruct((NC, M, H), jnp.float32),
              jax.ShapeDtypeStruct((NC, M, H), jnp.float32)],
    mesh=plsc.VectorSubcoreMesh(core_axis_name="c", subcore_axis_name="s"),
    scratch_types=[
        pltpu.VMEM((K,), jnp.int32),
        pltpu.VMEM((K,), jnp.int32),
        pltpu.VMEM((K, 2 * H), jnp.float32),
        pltpu.VMEM((K, H), jnp.float32),
        pltpu.VMEM((K, H), jnp.float32),
        pltpu.VMEM((K, H), jnp.float32),
        pltpu.VMEM((16, H), jnp.float32),
        pltpu.VMEM_SHARED((M, H), jnp.float32),
        pltpu.VMEM_SHARED((M, H), jnp.float32),
    ],
)(_edge_body)


def _epilogue_body(acc_ref, cnt_ref, wout_ref, bout_ref, out_ref):
    tot = acc_ref[0] + acc_ref[1]
    cnt = cnt_ref[0, :, 0:1] + cnt_ref[1, :, 0:1]
    pooled = tot / jnp.maximum(cnt, 1.0)
    out_ref[...] = jnp.dot(pooled, wout_ref[...],
                           preferred_element_type=jnp.float32) + bout_ref[...]


@jax.jit
def kernel(x, phys_pos, latent_tokens_pos, edge_index,
           W_lift, b_lift, W_k1, b_k1, W_out, b_out):
    u = W_k1[0:3] + W_k1[6:9]
    v = W_k1[3:6] - W_k1[6:9]
    u8 = jnp.concatenate([u, jnp.zeros((5, H), jnp.float32)], axis=0)
    v8 = jnp.concatenate([v, jnp.zeros((5, H), jnp.float32)], axis=0)
    p8 = jnp.pad(phys_pos, ((0, 0), (0, 5)))
    l8 = jnp.pad(latent_tokens_pos, ((0, 0), (0, 5)))
    src = edge_index[0].astype(jnp.int32)
    dst = edge_index[1].astype(jnp.int32)
    bl = b_lift.reshape(1, H)
    bk = b_k1.reshape(1, H)
    bo = b_out.reshape(1, D)

    rb = 2000
    srct = pl.pallas_call(
        _src_table_body,
        grid=(N // rb,),
        in_specs=[
            pl.BlockSpec((rb, D), lambda i: (i, 0)),
            pl.BlockSpec((rb, 8), lambda i: (i, 0)),
            pl.BlockSpec((8, H), lambda i: (0, 0)),
            pl.BlockSpec((D, H), lambda i: (0, 0)),
            pl.BlockSpec((1, H), lambda i: (0, 0)),
        ],
        out_specs=pl.BlockSpec((rb, 2 * H), lambda i: (i, 0)),
        out_shape=jax.ShapeDtypeStruct((N, 2 * H), jnp.float32),
    )(x, p8, u8, W_lift, bl)

    dstt = pl.pallas_call(
        _dst_table_body,
        out_shape=jax.ShapeDtypeStruct((M, H), jnp.float32),
    )(l8, v8, bk)

    acc, cnt = _edge_kernel(srct, dstt, src, dst)

    out = pl.pallas_call(
        _epilogue_body,
        out_shape=jax.ShapeDtypeStruct((M, D), jnp.float32),
    )(acc, cnt, W_out, bo)
    return out
